# baseline (speedup 1.0000x reference)
import functools

import jax
import jax.numpy as jnp
from jax import lax
from jax.experimental import pallas as pl
from jax.experimental.pallas import tpu as pltpu
from jax.experimental.pallas import tpu_sc as plsc

N = 10000
E = 320000
D = 128
DW = D // 2
DE = 144
NC = 2
NS = 16
NW = NC * NS
K = 128
CPT = 80
EPT = CPT * K
EPAD = NW * EPT
RPS = N // NS
BLK = 1000


def _mm_body(x_ref, w_ref, a_ref, he_ref, sa_ref):
    x = x_ref[...]
    h = jnp.dot(x, w_ref[...], precision=lax.Precision.HIGHEST)
    sa = jnp.dot(h, a_ref[...], precision=lax.Precision.HIGHEST)
    hh = h.reshape(x.shape[0], D // 32, 2, 16)
    hperm = jnp.transpose(hh, (0, 1, 3, 2)).reshape(x.shape[0], D)
    he_ref[...] = hperm.astype(jnp.bfloat16)
    sa_ref[...] = sa


_mm = pl.pallas_call(
    _mm_body,
    grid=(N // BLK,),
    in_specs=[
        pl.BlockSpec((BLK, D), lambda i: (i, jnp.int32(0))),
        pl.BlockSpec((D, D), lambda i: (jnp.int32(0), jnp.int32(0))),
        pl.BlockSpec((D, 8), lambda i: (jnp.int32(0), jnp.int32(0))),
    ],
    out_specs=[
        pl.BlockSpec((BLK, D), lambda i: (i, jnp.int32(0))),
        pl.BlockSpec((BLK, 8), lambda i: (i, jnp.int32(0))),
    ],
    out_shape=[
        jax.ShapeDtypeStruct((N, D), jnp.bfloat16),
        jax.ShapeDtypeStruct((N, 8), jnp.float32),
    ],
)


def _comb_body(acc_ref, b_ref, o_ref):
    acc = acc_ref[0] + acc_ref[1]
    den = acc[:, D:D + 1]
    x = acc[:, :D] / (den + 1e-16) + b_ref[...]
    o_ref[...] = jnp.maximum(x, 0.0)


_comb = pl.pallas_call(
    _comb_body,
    grid=(N // BLK,),
    in_specs=[
        pl.BlockSpec((2, BLK, DE), lambda i: (jnp.int32(0), i, jnp.int32(0))),
        pl.BlockSpec((1, D), lambda i: (jnp.int32(0), jnp.int32(0))),
    ],
    out_specs=pl.BlockSpec((BLK, D), lambda i: (i, jnp.int32(0))),
    out_shape=jax.ShapeDtypeStruct((N, D), jnp.float32),
)


_mesh = plsc.VectorSubcoreMesh(core_axis_name="c", subcore_axis_name="s")


@functools.partial(
    pl.kernel,
    mesh=_mesh,
    compiler_params=pltpu.CompilerParams(use_tc_tiling_on_sc=False,
                                         needs_layout_passes=False),
    out_type=jax.ShapeDtypeStruct((NC * N, DE), jnp.float32),
    scratch_types=[
        pltpu.VMEM((8, K), jnp.int32),
        pltpu.VMEM((2, K), jnp.float32),
        pltpu.VMEM((2, K), jnp.float32),
        pltpu.VMEM((K,), jnp.float32),
        pltpu.VMEM((K, DW), jnp.int32),
        pltpu.VMEM((K, DW), jnp.int32),
        pltpu.VMEM((K, DE), jnp.float32),
        pltpu.VMEM_SHARED((N,), jnp.float32),
        pltpu.VMEM_SHARED((N,), jnp.float32),
        pltpu.VMEM_SHARED((N, DE), jnp.float32),
        pltpu.SemaphoreType.DMA,
        pltpu.SemaphoreType.DMA,
        pltpu.SemaphoreType.DMA,
        pltpu.SemaphoreType.DMA,
        pltpu.SemaphoreType.DMA,
    ],
)
def _sc_edges(h_hbm, as_hbm, ad_hbm, sd_hbm, acc_hbm,
              sd_c, asg_v, adg_v, w_c, rows0, rows1, stage,
              as_sh, ad_sh, acc_sh,
              sem_g, sem_s, sem_a, sem_d, sem_sd):
    c = lax.axis_index("c")
    s = lax.axis_index("s")
    wid = c * jnp.int32(NS) + s

    @pl.when(s == jnp.int32(0))
    def _():
        pltpu.sync_copy(as_hbm, as_sh)
        pltpu.sync_copy(ad_hbm, ad_sh)

    zero16 = jnp.zeros((16,), jnp.float32)

    def zr(r, carry):
        for j in range(DE // 16):
            stage[r, pl.ds(j * 16, 16)] = zero16
        return carry

    lax.fori_loop(jnp.int32(0), jnp.int32(K), zr, jnp.int32(0))
    rbase = s * jnp.int32(RPS)
    off = 0
    for sz in (128, 128, 128, 128, RPS - 512):
        pltpu.sync_copy(stage.at[pl.ds(0, sz)],
                        acc_sh.at[pl.ds(rbase + off, sz)])
        off += sz
    plsc.subcore_barrier()

    ebase = wid * jnp.int32(EPT)
    lane = lax.iota(jnp.int32, 16)
    rows = (rows0, rows1)
    row20 = (wid * jnp.int32(CPT)) * jnp.int32(2)

    pltpu.sync_copy(sd_hbm.at[pl.ds(row20, 2)], sd_c.at[pl.ds(0, 2)])
    pltpu.async_copy(sd_hbm.at[pl.ds(row20 + jnp.int32(2), 2)],
                     sd_c.at[pl.ds(2, 2)], sem_sd)
    pltpu.async_copy(h_hbm.at[sd_c.at[jnp.int32(0)]], rows0, sem_g)
    pltpu.async_copy(as_sh.at[sd_c.at[jnp.int32(0)]],
                     asg_v.at[jnp.int32(0)], sem_a)
    pltpu.async_copy(ad_sh.at[sd_c.at[jnp.int32(1)]],
                     adg_v.at[jnp.int32(0)], sem_d)

    def chunk(i, b):
        p, q = b % 2, 1 - (b % 2)
        bn, bn2 = (b + 1) % 4, (b + 2) % 4
        row2 = row20 + i * jnp.int32(2)

        @pl.when(i + jnp.int32(2) < jnp.int32(CPT))
        def _():
            pltpu.async_copy(sd_hbm.at[pl.ds(row2 + jnp.int32(4), 2)],
                             sd_c.at[pl.ds(2 * bn2, 2)], sem_sd)

        @pl.when(i + jnp.int32(1) < jnp.int32(CPT))
        def _():
            pltpu.make_async_copy(sd_hbm.at[pl.ds(row2 + jnp.int32(2), 2)],
                                  sd_c.at[pl.ds(2 * bn, 2)], sem_sd).wait()
            pltpu.async_copy(h_hbm.at[sd_c.at[jnp.int32(2 * bn)]], rows[q],
                             sem_g)
            pltpu.async_copy(as_sh.at[sd_c.at[jnp.int32(2 * bn)]],
                             asg_v.at[jnp.int32(q)], sem_a)
            pltpu.async_copy(ad_sh.at[sd_c.at[jnp.int32(2 * bn + 1)]],
                             adg_v.at[jnp.int32(q)], sem_d)

        pltpu.make_async_copy(as_sh.at[sd_c.at[jnp.int32(2 * b)]],
                              asg_v.at[jnp.int32(p)], sem_a).wait()
        pltpu.make_async_copy(ad_sh.at[sd_c.at[jnp.int32(2 * b + 1)]],
                              adg_v.at[jnp.int32(p)], sem_d).wait()
        for j in range(K // 16):
            e = asg_v[p, pl.ds(j * 16, 16)] + adg_v[p, pl.ds(j * 16, 16)]
            e = jnp.where(e >= 0.0, e, 0.2 * e)
            eid = ebase + i * jnp.int32(K) + jnp.int32(j * 16) + lane
            w_c[pl.ds(j * 16, 16)] = jnp.where(eid < jnp.int32(E),
                                               jnp.exp(e), 0.0)

        pltpu.make_async_copy(h_hbm.at[sd_c.at[jnp.int32(2 * b)]], rows[p],
                              sem_g).wait()

        @pl.when(i > jnp.int32(0))
        def _():
            pltpu.make_async_copy(
                stage, acc_sh.at[sd_c.at[jnp.int32(2 * ((b + 3) % 4) + 1)]],
                sem_s).wait()

        e0 = jnp.where(lane == jnp.int32(0), jnp.float32(1.0),
                       jnp.float32(0.0))

        def rblock(rb, inner):
            w16 = w_c[pl.ds(rb * 16, 16)]
            for l in range(16):
                r = rb * jnp.int32(16) + jnp.int32(l)
                wb = lax.gather(
                    w16, jnp.full((16, 1), l, jnp.int32),
                    dimension_numbers=lax.GatherDimensionNumbers(
                        offset_dims=(), collapsed_slice_dims=(0,),
                        start_index_map=(0,)),
                    slice_sizes=(1,),
                    mode=lax.GatherScatterMode.PROMISE_IN_BOUNDS)
                for j4 in range(D // 32):
                    vi = rows[p][r, pl.ds(j4 * 16, 16)]
                    vb = plsc.bitcast(vi, jnp.bfloat16)
                    va, vc = plsc.unpack(vb, format=plsc.PackFormat.INTERLEAVED)
                    stage[r, pl.ds(j4 * 32, 16)] = va * wb
                    stage[r, pl.ds(j4 * 32 + 16, 16)] = vc * wb
                stage[r, pl.ds(D, 16)] = wb * e0
            return inner

        lax.fori_loop(jnp.int32(0), jnp.int32(K // 16), rblock, jnp.int32(0))
        pltpu.async_copy(stage, acc_sh.at[sd_c.at[jnp.int32(2 * b + 1)]],
                         sem_s, add=True)

    def group(g, carry):
        for b in range(4):
            chunk(g * jnp.int32(4) + jnp.int32(b), b)
        return carry

    lax.fori_loop(jnp.int32(0), jnp.int32(CPT // 4), group, jnp.int32(0))
    pltpu.make_async_copy(stage, acc_sh.at[sd_c.at[jnp.int32(7)]],
                          sem_s).wait()

    plsc.subcore_barrier()
    pltpu.sync_copy(acc_sh.at[pl.ds(rbase, RPS)],
                    acc_hbm.at[pl.ds(c * jnp.int32(N) + rbase, RPS)])


def _layer(xin, W, a_s, a_d, b, sdr):
    A = jnp.zeros((D, 8), jnp.float32).at[:, 0].set(a_s).at[:, 1].set(a_d)
    he, sa = _mm(xin, W, A)
    h_i32 = lax.bitcast_convert_type(he.reshape(N, DW, 2), jnp.int32)
    accp = _sc_edges(h_i32, sa[:, 0], sa[:, 1], sdr)
    return _comb(accp.reshape(2, N, DE), b.reshape(1, D))


def kernel(x, g, W1, a_s1, a_d1, b1, W2, a_s2, a_d2, b2):
    src = g[0].astype(jnp.int32)
    dst = g[1].astype(jnp.int32)
    srcr = jnp.pad(src, (0, EPAD - E)).reshape(NW * CPT, K)
    dstr = jnp.pad(dst, (0, EPAD - E)).reshape(NW * CPT, K)
    sdr = jnp.stack([srcr, dstr], axis=1).reshape(NW * CPT * 2, K)
    x1 = _layer(x, W1, a_s1, a_d1, b1, sdr)
    return _layer(x1, W2, a_s2, a_d2, b2, sdr)


# speedup vs baseline: 1.0523x; 1.0523x over previous
import functools

import jax
import jax.numpy as jnp
from jax import lax
from jax.experimental import pallas as pl
from jax.experimental.pallas import tpu as pltpu
from jax.experimental.pallas import tpu_sc as plsc

N = 10000
E = 320000
D = 128
DW = D // 2
NC = 2
NS = 16
NW = NC * NS
K = 128
CPT = 80
EPT = CPT * K
EPAD = NW * EPT
RPS = N // NS
NDEN = 10240
DPS = NDEN // NS
BLK = 1000


def _mm_body(x_ref, w_ref, a_ref, he_ref, sa_ref):
    x = x_ref[...]
    h = jnp.dot(x, w_ref[...], precision=lax.Precision.HIGHEST)
    sa = jnp.dot(h, a_ref[...], precision=lax.Precision.HIGHEST)
    hh = h.reshape(x.shape[0], D // 32, 2, 16)
    hperm = jnp.transpose(hh, (0, 1, 3, 2)).reshape(x.shape[0], D)
    he_ref[...] = hperm.astype(jnp.bfloat16)
    sa_ref[...] = sa


_mm = pl.pallas_call(
    _mm_body,
    grid=(N // BLK,),
    in_specs=[
        pl.BlockSpec((BLK, D), lambda i: (i, jnp.int32(0))),
        pl.BlockSpec((D, D), lambda i: (jnp.int32(0), jnp.int32(0))),
        pl.BlockSpec((D, 8), lambda i: (jnp.int32(0), jnp.int32(0))),
    ],
    out_specs=[
        pl.BlockSpec((BLK, D), lambda i: (i, jnp.int32(0))),
        pl.BlockSpec((BLK, 8), lambda i: (i, jnp.int32(0))),
    ],
    out_shape=[
        jax.ShapeDtypeStruct((N, D), jnp.bfloat16),
        jax.ShapeDtypeStruct((N, 8), jnp.float32),
    ],
)


def _comb_body(acc_ref, den_ref, b_ref, o_ref):
    acc = acc_ref[0] + acc_ref[1]
    den = den_ref[:, 0] + den_ref[:, 1]
    x = acc / (den[:, None] + 1e-16) + b_ref[...]
    o_ref[...] = jnp.maximum(x, 0.0)


_comb = pl.pallas_call(
    _comb_body,
    grid=(N // BLK,),
    in_specs=[
        pl.BlockSpec((2, BLK, D), lambda i: (jnp.int32(0), i, jnp.int32(0))),
        pl.BlockSpec((BLK, 2), lambda i: (i, jnp.int32(0))),
        pl.BlockSpec((1, D), lambda i: (jnp.int32(0), jnp.int32(0))),
    ],
    out_specs=pl.BlockSpec((BLK, D), lambda i: (i, jnp.int32(0))),
    out_shape=jax.ShapeDtypeStruct((N, D), jnp.float32),
)


_mesh = plsc.VectorSubcoreMesh(core_axis_name="c", subcore_axis_name="s")


@functools.partial(
    pl.kernel,
    mesh=_mesh,
    compiler_params=pltpu.CompilerParams(use_tc_tiling_on_sc=False,
                                         needs_layout_passes=False),
    out_type=[
        jax.ShapeDtypeStruct((NC * N, D), jnp.float32),
        jax.ShapeDtypeStruct((NC, NDEN), jnp.float32),
    ],
    scratch_types=[
        pltpu.VMEM((8, K), jnp.int32),
        pltpu.VMEM((2, K), jnp.float32),
        pltpu.VMEM((2, K), jnp.float32),
        pltpu.VMEM((2, K), jnp.float32),
        pltpu.VMEM((K, DW), jnp.int32),
        pltpu.VMEM((K, DW), jnp.int32),
        pltpu.VMEM((K, D), jnp.float32),
        pltpu.VMEM_SHARED((N,), jnp.float32),
        pltpu.VMEM_SHARED((N,), jnp.float32),
        pltpu.VMEM_SHARED((NDEN,), jnp.float32),
        pltpu.VMEM_SHARED((N, D), jnp.float32),
        pltpu.SemaphoreType.DMA,
        pltpu.SemaphoreType.DMA,
        pltpu.SemaphoreType.DMA,
        pltpu.SemaphoreType.DMA,
        pltpu.SemaphoreType.DMA,
        pltpu.SemaphoreType.DMA,
    ],
)
def _sc_edges(h_hbm, as_hbm, ad_hbm, sd_hbm, acc_hbm, den_hbm,
              sd_c, asg_v, adg_v, w_c, rows0, rows1, stage,
              as_sh, ad_sh, den_sh, acc_sh,
              sem_g, sem_s, sem_a, sem_d, sem_w, sem_sd):
    c = lax.axis_index("c")
    s = lax.axis_index("s")
    wid = c * jnp.int32(NS) + s

    @pl.when(s == jnp.int32(0))
    def _():
        pltpu.sync_copy(as_hbm, as_sh)
        pltpu.sync_copy(ad_hbm, ad_sh)

    zero16 = jnp.zeros((16,), jnp.float32)

    def zr(r, carry):
        for j in range(D // 16):
            stage[r, pl.ds(j * 16, 16)] = zero16
        return carry

    lax.fori_loop(jnp.int32(0), jnp.int32(K), zr, jnp.int32(0))
    rbase = s * jnp.int32(RPS)
    off = 0
    for sz in (128, 128, 128, 128, RPS - 512):
        pltpu.sync_copy(stage.at[pl.ds(0, sz)],
                        acc_sh.at[pl.ds(rbase + off, sz)])
        off += sz
    dbase = s * jnp.int32(DPS)
    for k in range(DPS // K):
        pltpu.sync_copy(stage.at[jnp.int32(0)],
                        den_sh.at[pl.ds(dbase + jnp.int32(k * K), K)])
    plsc.subcore_barrier()

    ebase = wid * jnp.int32(EPT)
    lane = lax.iota(jnp.int32, 16)
    rows = (rows0, rows1)
    row20 = (wid * jnp.int32(CPT)) * jnp.int32(2)

    pltpu.sync_copy(sd_hbm.at[pl.ds(row20, 2)], sd_c.at[pl.ds(0, 2)])
    pltpu.async_copy(sd_hbm.at[pl.ds(row20 + jnp.int32(2), 2)],
                     sd_c.at[pl.ds(2, 2)], sem_sd)
    pltpu.async_copy(h_hbm.at[sd_c.at[jnp.int32(0)]], rows0, sem_g)
    pltpu.async_copy(as_sh.at[sd_c.at[jnp.int32(0)]],
                     asg_v.at[jnp.int32(0)], sem_a)
    pltpu.async_copy(ad_sh.at[sd_c.at[jnp.int32(1)]],
                     adg_v.at[jnp.int32(0)], sem_d)

    def chunk(i, b):
        p, q = b % 2, 1 - (b % 2)
        bn, bn2 = (b + 1) % 4, (b + 2) % 4
        row2 = row20 + i * jnp.int32(2)

        @pl.when(i + jnp.int32(2) < jnp.int32(CPT))
        def _():
            pltpu.async_copy(sd_hbm.at[pl.ds(row2 + jnp.int32(4), 2)],
                             sd_c.at[pl.ds(2 * bn2, 2)], sem_sd)

        @pl.when(i + jnp.int32(1) < jnp.int32(CPT))
        def _():
            pltpu.make_async_copy(sd_hbm.at[pl.ds(row2 + jnp.int32(2), 2)],
                                  sd_c.at[pl.ds(2 * bn, 2)], sem_sd).wait()
            pltpu.async_copy(h_hbm.at[sd_c.at[jnp.int32(2 * bn)]], rows[q],
                             sem_g)
            pltpu.async_copy(as_sh.at[sd_c.at[jnp.int32(2 * bn)]],
                             asg_v.at[jnp.int32(q)], sem_a)
            pltpu.async_copy(ad_sh.at[sd_c.at[jnp.int32(2 * bn + 1)]],
                             adg_v.at[jnp.int32(q)], sem_d)

        @pl.when(i > jnp.int32(1))
        def _():
            pltpu.make_async_copy(
                w_c.at[jnp.int32(p)],
                den_sh.at[sd_c.at[jnp.int32(2 * b + 1)]], sem_w).wait()

        pltpu.make_async_copy(as_sh.at[sd_c.at[jnp.int32(2 * b)]],
                              asg_v.at[jnp.int32(p)], sem_a).wait()
        pltpu.make_async_copy(ad_sh.at[sd_c.at[jnp.int32(2 * b + 1)]],
                              adg_v.at[jnp.int32(p)], sem_d).wait()
        for j in range(K // 16):
            e = asg_v[p, pl.ds(j * 16, 16)] + adg_v[p, pl.ds(j * 16, 16)]
            e = jnp.where(e >= 0.0, e, 0.2 * e)
            eid = ebase + i * jnp.int32(K) + jnp.int32(j * 16) + lane
            w_c[p, pl.ds(j * 16, 16)] = jnp.where(eid < jnp.int32(E),
                                                  jnp.exp(e), 0.0)
        pltpu.async_copy(w_c.at[jnp.int32(p)],
                         den_sh.at[sd_c.at[jnp.int32(2 * b + 1)]], sem_w,
                         add=True)

        pltpu.make_async_copy(h_hbm.at[sd_c.at[jnp.int32(2 * b)]], rows[p],
                              sem_g).wait()

        @pl.when(i > jnp.int32(0))
        def _():
            pltpu.make_async_copy(
                stage, acc_sh.at[sd_c.at[jnp.int32(2 * ((b + 3) % 4) + 1)]],
                sem_s).wait()

        def rblock(rb, inner):
            w16 = w_c[p, pl.ds(rb * 16, 16)]
            for l in range(16):
                r = rb * jnp.int32(16) + jnp.int32(l)
                wb = lax.gather(
                    w16, jnp.full((16, 1), l, jnp.int32),
                    dimension_numbers=lax.GatherDimensionNumbers(
                        offset_dims=(), collapsed_slice_dims=(0,),
                        start_index_map=(0,)),
                    slice_sizes=(1,),
                    mode=lax.GatherScatterMode.PROMISE_IN_BOUNDS)
                for j4 in range(D // 32):
                    vi = rows[p][r, pl.ds(j4 * 16, 16)]
                    vb = plsc.bitcast(vi, jnp.bfloat16)
                    va, vc = plsc.unpack(vb, format=plsc.PackFormat.INTERLEAVED)
                    stage[r, pl.ds(j4 * 32, 16)] = va * wb
                    stage[r, pl.ds(j4 * 32 + 16, 16)] = vc * wb
            return inner

        lax.fori_loop(jnp.int32(0), jnp.int32(K // 16), rblock, jnp.int32(0))
        pltpu.async_copy(stage, acc_sh.at[sd_c.at[jnp.int32(2 * b + 1)]],
                         sem_s, add=True)

    def group(g, carry):
        for b in range(4):
            chunk(g * jnp.int32(4) + jnp.int32(b), b)
        return carry

    lax.fori_loop(jnp.int32(0), jnp.int32(CPT // 4), group, jnp.int32(0))
    pltpu.make_async_copy(stage, acc_sh.at[sd_c.at[jnp.int32(7)]],
                          sem_s).wait()
    pltpu.make_async_copy(w_c.at[jnp.int32(0)],
                          den_sh.at[sd_c.at[jnp.int32(5)]], sem_w).wait()
    pltpu.make_async_copy(w_c.at[jnp.int32(1)],
                          den_sh.at[sd_c.at[jnp.int32(7)]], sem_w).wait()

    plsc.subcore_barrier()
    pltpu.sync_copy(acc_sh.at[pl.ds(rbase, RPS)],
                    acc_hbm.at[pl.ds(c * jnp.int32(N) + rbase, RPS)])
    pltpu.sync_copy(den_sh.at[pl.ds(dbase, DPS)],
                    den_hbm.at[c, pl.ds(dbase, DPS)])


def _layer(xin, W, a_s, a_d, b, sdr):
    A = jnp.zeros((D, 8), jnp.float32).at[:, 0].set(a_s).at[:, 1].set(a_d)
    he, sa = _mm(xin, W, A)
    h_i32 = lax.bitcast_convert_type(he.reshape(N, DW, 2), jnp.int32)
    accp, denp = _sc_edges(h_i32, sa[:, 0], sa[:, 1], sdr)
    return _comb(accp.reshape(2, N, D), denp[:, :N].T, b.reshape(1, D))


def kernel(x, g, W1, a_s1, a_d1, b1, W2, a_s2, a_d2, b2):
    src = g[0].astype(jnp.int32)
    dst = g[1].astype(jnp.int32)
    srcr = jnp.pad(src, (0, EPAD - E)).reshape(NW * CPT, K)
    dstr = jnp.pad(dst, (0, EPAD - E)).reshape(NW * CPT, K)
    sdr = jnp.stack([srcr, dstr], axis=1).reshape(NW * CPT * 2, K)
    x1 = _layer(x, W1, a_s1, a_d1, b1, sdr)
    return _layer(x1, W2, a_s2, a_d2, b2, sdr)
